# Initial kernel scaffold; baseline (speedup 1.0000x reference)
import jax
import jax.numpy as jnp
from jax.experimental import pallas as pl


def kernel(num_obj, object_branch_output, W_self, W_neigh, b):
    raise NotImplementedError("write your pallas kernel here")


# speedup vs baseline: 3.1146x; 3.1146x over previous
import dataclasses
import functools

import jax
import jax.numpy as jnp
from jax import lax
from jax.experimental import pallas as pl
from jax.experimental.pallas import tpu as pltpu
from jax.experimental.pallas import tpu_sc as plsc

_NW = 32
_LANES = 16


def _sc_seg_stats(starts, ends, x, kmax):
    B = ends.shape[0]
    N, D = x.shape
    TB = B // _NW
    TR = N // _NW
    CB = 32
    WIN = CB * kmax + 8
    NCH = TB // CB
    NC = D // _LANES
    NG = TR // _LANES

    mesh = plsc.VectorSubcoreMesh(core_axis_name="c", subcore_axis_name="s")
    cp = pltpu.CompilerParams()
    if "needs_layout_passes" in pltpu.CompilerParams.__dataclass_fields__:
        cp = dataclasses.replace(cp, needs_layout_passes=False)

    @functools.partial(
        pl.kernel,
        mesh=mesh,
        compiler_params=cp,
        out_type=(jax.ShapeDtypeStruct((B, D), jnp.float32),
                  jax.ShapeDtypeStruct((N,), jnp.int32)),
        scratch_types=[
            pltpu.VMEM((WIN, D), jnp.float32),
            pltpu.VMEM((CB, D), jnp.float32),
            pltpu.VMEM((B + 16,), jnp.int32),
            pltpu.VMEM((TB + 16,), jnp.int32),
            pltpu.VMEM((TR,), jnp.int32),
            pltpu.SemaphoreType.DMA,
        ],
    )
    def k(starts_hbm, ends_hbm, x_hbm, m_hbm, rs_hbm,
          xw, mbuf, ends_v, starts_v, segbuf, sem):
        w = lax.axis_index("s") * 2 + lax.axis_index("c")
        pltpu.sync_copy(ends_hbm, ends_v.at[pl.ds(0, B)])
        pltpu.sync_copy(starts_hbm.at[pl.ds(w * TB, TB)], starts_v.at[pl.ds(0, TB)])

        def sload(ref, i):
            return ref[pl.ds(i, _LANES)][0]

        @pl.loop(0, NCH)
        def _(ch):
            c0 = ch * CB
            s0 = jnp.minimum((sload(starts_v, c0) // 8) * 8, N - WIN)
            pltpu.async_copy(x_hbm.at[pl.ds(s0, WIN)], xw, sem).wait()

            @pl.loop(0, CB)
            def _(bi):
                sti = sload(starts_v, c0 + bi)
                st = sti - s0
                kk = sload(ends_v, w * TB + c0 + bi) - sti
                accs = [jnp.zeros((_LANES,), jnp.float32) for _ in range(NC)]
                for a in range(kmax):
                    valid = (a < kk).astype(jnp.float32)
                    for c in range(NC):
                        accs[c] = accs[c] + xw[st + a, pl.ds(c * _LANES, _LANES)] * valid
                inv = jnp.float32(1.0)
                for kv in range(2, kmax + 1):
                    inv = jnp.where(kk == kv, jnp.float32(1.0 / kv), inv)
                for c in range(NC):
                    mbuf[bi, pl.ds(c * _LANES, _LANES)] = accs[c] * inv

            pltpu.async_copy(mbuf, m_hbm.at[pl.ds(w * TB + c0, CB)], sem).wait()

        r0 = w * TR
        lane = lax.iota(jnp.int32, 16)
        total_v = plsc.load_gather(ends_v, [jnp.full((16,), B - 1, jnp.int32)])
        bcast_B = jnp.full((16,), B, jnp.int32)

        @pl.loop(0, NG)
        def _(g):
            rvec = r0 + g * _LANES + lane

            def bs(_, lohi):
                lo, hi = lohi
                mid = jnp.minimum((lo + hi) >> 1, B - 1)
                ev = plsc.load_gather(ends_v, [mid])
                gt = ev > rvec
                return (jnp.where(gt, lo, mid + 1), jnp.where(gt, mid, hi))

            lo, _hi = lax.fori_loop(
                0, 15, bs, (jnp.zeros((16,), jnp.int32), bcast_B))
            segbuf[pl.ds(g * _LANES, _LANES)] = jnp.where(
                rvec < total_v, lo, bcast_B)

        pltpu.sync_copy(segbuf, rs_hbm.at[pl.ds(r0, TR)])

    return k(starts, ends, x)


def _tc_matmul(m, w_neigh):
    B, D = m.shape
    BLK = 512

    def body(m_ref, w_ref, y_ref):
        y_ref[...] = jnp.dot(m_ref[...], w_ref[...],
                             preferred_element_type=jnp.float32)

    return pl.pallas_call(
        body,
        grid=(B // BLK,),
        in_specs=[pl.BlockSpec((BLK, D), lambda j: (j, 0)),
                  pl.BlockSpec((D, D), lambda j: (0, 0))],
        out_specs=pl.BlockSpec((BLK, D), lambda j: (j, 0)),
        out_shape=jax.ShapeDtypeStruct((B, D), jnp.float32),
    )(m, w_neigh)


def _sc_expand(y_pad, row_seg, n_rows):
    D = y_pad.shape[1]
    N = n_rows
    TR = N // _NW
    RC = 256
    NCH = TR // RC

    mesh = plsc.VectorSubcoreMesh(core_axis_name="c", subcore_axis_name="s")
    cp = pltpu.CompilerParams()
    if "needs_layout_passes" in pltpu.CompilerParams.__dataclass_fields__:
        cp = dataclasses.replace(cp, needs_layout_passes=False)

    @functools.partial(
        pl.kernel,
        mesh=mesh,
        compiler_params=cp,
        out_type=jax.ShapeDtypeStruct((N, D), jnp.float32),
        scratch_types=[
            pltpu.VMEM((RC,), jnp.int32),
            pltpu.VMEM((RC, D), jnp.float32),
            pltpu.SemaphoreType.DMA,
        ],
    )
    def k(y_hbm, rs_hbm, ya_hbm, idx_v, rows_v, sem):
        w = lax.axis_index("s") * 2 + lax.axis_index("c")
        r0 = w * TR

        @pl.loop(0, NCH)
        def _(ci):
            base = r0 + ci * RC
            pltpu.sync_copy(rs_hbm.at[pl.ds(base, RC)], idx_v)
            pltpu.async_copy(y_hbm.at[idx_v], rows_v, sem).wait()
            pltpu.sync_copy(rows_v, ya_hbm.at[pl.ds(base, RC)])

    return k(y_pad, row_seg)


def _tc_final(x, w_self, b, ya):
    N, D = x.shape
    BLK = 512

    def body(x_ref, w_ref, b_ref, ya_ref, o_ref):
        z = jnp.dot(x_ref[...], w_ref[...], preferred_element_type=jnp.float32)
        o_ref[...] = jnp.maximum(z + ya_ref[...] + b_ref[...], 0.0)

    return pl.pallas_call(
        body,
        grid=(N // BLK,),
        in_specs=[pl.BlockSpec((BLK, D), lambda j: (j, 0)),
                  pl.BlockSpec((D, D), lambda j: (0, 0)),
                  pl.BlockSpec((1, D), lambda j: (0, 0)),
                  pl.BlockSpec((BLK, D), lambda j: (j, 0))],
        out_specs=pl.BlockSpec((BLK, D), lambda j: (j, 0)),
        out_shape=jax.ShapeDtypeStruct((N, D), jnp.float32),
    )(x, w_self, b.reshape(1, D), ya)


def kernel(num_obj, object_branch_output, W_self, W_neigh, b):
    x = object_branch_output
    N, D = x.shape
    B = num_obj.shape[0]
    kmax = N // B
    ends = jnp.cumsum(num_obj.astype(jnp.int32))
    starts = ends - num_obj.astype(jnp.int32)
    m, row_seg = _sc_seg_stats(starts, ends, x, kmax)
    y = _tc_matmul(m, W_neigh)
    y_pad = jnp.concatenate([y, jnp.zeros((8, D), jnp.float32)], axis=0)
    ya = _sc_expand(y_pad, row_seg, N)
    return _tc_final(x, W_self, b, ya)


# speedup vs baseline: 3.1211x; 1.0021x over previous
import dataclasses
import functools

import jax
import jax.numpy as jnp
from jax import lax
from jax.experimental import pallas as pl
from jax.experimental.pallas import tpu as pltpu
from jax.experimental.pallas import tpu_sc as plsc

_NW = 32
_LANES = 16


def _sc_seg_stats(starts, ends, x, kmax):
    B = ends.shape[0]
    N, D = x.shape
    TB = B // _NW
    TR = N // _NW
    CB = 32
    WIN = CB * kmax + 8
    NCH = TB // CB
    NC = D // _LANES
    NG = TR // _LANES

    mesh = plsc.VectorSubcoreMesh(core_axis_name="c", subcore_axis_name="s")
    cp = pltpu.CompilerParams()
    if "needs_layout_passes" in pltpu.CompilerParams.__dataclass_fields__:
        cp = dataclasses.replace(cp, needs_layout_passes=False)

    @functools.partial(
        pl.kernel,
        mesh=mesh,
        compiler_params=cp,
        out_type=(jax.ShapeDtypeStruct((B, D), jnp.float32),
                  jax.ShapeDtypeStruct((N,), jnp.int32)),
        scratch_types=[
            pltpu.VMEM((WIN, D), jnp.float32),
            pltpu.VMEM((CB, D), jnp.float32),
            pltpu.VMEM((B + 16,), jnp.int32),
            pltpu.VMEM((TB + 16,), jnp.int32),
            pltpu.VMEM((TR,), jnp.int32),
            pltpu.SemaphoreType.DMA,
        ],
    )
    def k(starts_hbm, ends_hbm, x_hbm, m_hbm, rs_hbm,
          xw, mbuf, ends_v, starts_v, segbuf, sem):
        w = lax.axis_index("s") * 2 + lax.axis_index("c")
        pltpu.sync_copy(ends_hbm, ends_v.at[pl.ds(0, B)])
        pltpu.sync_copy(starts_hbm.at[pl.ds(w * TB, TB)], starts_v.at[pl.ds(0, TB)])

        def sload(ref, i):
            return ref[pl.ds(i, _LANES)][0]

        @pl.loop(0, NCH)
        def _(ch):
            c0 = ch * CB
            s0 = jnp.minimum((sload(starts_v, c0) // 8) * 8, N - WIN)
            pltpu.async_copy(x_hbm.at[pl.ds(s0, WIN)], xw, sem).wait()

            @pl.loop(0, CB)
            def _(bi):
                sti = sload(starts_v, c0 + bi)
                st = sti - s0
                kk = sload(ends_v, w * TB + c0 + bi) - sti
                accs = [jnp.zeros((_LANES,), jnp.float32) for _ in range(NC)]
                for a in range(kmax):
                    valid = (a < kk).astype(jnp.float32)
                    for c in range(NC):
                        accs[c] = accs[c] + xw[st + a, pl.ds(c * _LANES, _LANES)] * valid
                inv = jnp.float32(1.0)
                for kv in range(2, kmax + 1):
                    inv = jnp.where(kk == kv, jnp.float32(1.0 / kv), inv)
                for c in range(NC):
                    mbuf[bi, pl.ds(c * _LANES, _LANES)] = accs[c] * inv

            pltpu.async_copy(mbuf, m_hbm.at[pl.ds(w * TB + c0, CB)], sem).wait()

        r0 = w * TR
        lane = lax.iota(jnp.int32, 16)
        total_v = plsc.load_gather(ends_v, [jnp.full((16,), B - 1, jnp.int32)])
        bcast_B = jnp.full((16,), B, jnp.int32)

        @pl.loop(0, NG)
        def _(g):
            rvec = r0 + g * _LANES + lane

            def bs(_, lohi):
                lo, hi = lohi
                mid = jnp.minimum((lo + hi) >> 1, B - 1)
                ev = plsc.load_gather(ends_v, [mid])
                gt = ev > rvec
                return (jnp.where(gt, lo, mid + 1), jnp.where(gt, mid, hi))

            lo, _hi = lax.fori_loop(
                0, 15, bs, (jnp.zeros((16,), jnp.int32), bcast_B))
            segbuf[pl.ds(g * _LANES, _LANES)] = jnp.where(
                rvec < total_v, lo, bcast_B)

        pltpu.sync_copy(segbuf, rs_hbm.at[pl.ds(r0, TR)])

    return k(starts, ends, x)


def _tc_matmul(m, w_neigh):
    B, D = m.shape
    BLK = 512

    def body(m_ref, w_ref, y_ref):
        y_ref[...] = jnp.dot(m_ref[...], w_ref[...],
                             preferred_element_type=jnp.float32)

    return pl.pallas_call(
        body,
        grid=(B // BLK,),
        in_specs=[pl.BlockSpec((BLK, D), lambda j: (j, 0)),
                  pl.BlockSpec((D, D), lambda j: (0, 0))],
        out_specs=pl.BlockSpec((BLK, D), lambda j: (j, 0)),
        out_shape=jax.ShapeDtypeStruct((B, D), jnp.float32),
    )(m, w_neigh)


def _sc_expand(y_pad, row_seg, n_rows):
    D = y_pad.shape[1]
    N = n_rows
    TR = N // _NW
    RC = 256
    NCH = TR // RC

    mesh = plsc.VectorSubcoreMesh(core_axis_name="c", subcore_axis_name="s")
    cp = pltpu.CompilerParams()
    if "needs_layout_passes" in pltpu.CompilerParams.__dataclass_fields__:
        cp = dataclasses.replace(cp, needs_layout_passes=False)

    @functools.partial(
        pl.kernel,
        mesh=mesh,
        compiler_params=cp,
        out_type=jax.ShapeDtypeStruct((N, D), jnp.float32),
    )
    def k(y_hbm, rs_hbm, ya_hbm):
        def body(i_vmem, o_vmem):
            pltpu.sync_copy(y_hbm.at[i_vmem.at[0]], o_vmem)

        pltpu.emit_pipeline(
            body,
            grid=(N // RC,),
            in_specs=[pl.BlockSpec((1, RC), lambda i: (0, i))],
            out_specs=[pl.BlockSpec((RC, D), lambda i: (i, 0))],
            core_axis_name=("c", "s"),
            dimension_semantics=(pltpu.PARALLEL,),
        )(rs_hbm, ya_hbm)

    return k(y_pad, row_seg.reshape(1, N))


def _tc_final(x, w_self, b, ya):
    N, D = x.shape
    BLK = 512

    def body(x_ref, w_ref, b_ref, ya_ref, o_ref):
        z = jnp.dot(x_ref[...], w_ref[...], preferred_element_type=jnp.float32)
        o_ref[...] = jnp.maximum(z + ya_ref[...] + b_ref[...], 0.0)

    return pl.pallas_call(
        body,
        grid=(N // BLK,),
        in_specs=[pl.BlockSpec((BLK, D), lambda j: (j, 0)),
                  pl.BlockSpec((D, D), lambda j: (0, 0)),
                  pl.BlockSpec((1, D), lambda j: (0, 0)),
                  pl.BlockSpec((BLK, D), lambda j: (j, 0))],
        out_specs=pl.BlockSpec((BLK, D), lambda j: (j, 0)),
        out_shape=jax.ShapeDtypeStruct((N, D), jnp.float32),
    )(x, w_self, b.reshape(1, D), ya)


def kernel(num_obj, object_branch_output, W_self, W_neigh, b):
    x = object_branch_output
    N, D = x.shape
    B = num_obj.shape[0]
    kmax = N // B
    ends = jnp.cumsum(num_obj.astype(jnp.int32))
    starts = ends - num_obj.astype(jnp.int32)
    m, row_seg = _sc_seg_stats(starts, ends, x, kmax)
    y = _tc_matmul(m, W_neigh)
    y_pad = jnp.concatenate([y, jnp.zeros((8, D), jnp.float32)], axis=0)
    ya = _sc_expand(y_pad, row_seg, N)
    return _tc_final(x, W_self, b, ya)


# speedup vs baseline: 3.2502x; 1.0414x over previous
import dataclasses
import functools

import jax
import jax.numpy as jnp
from jax import lax
from jax.experimental import pallas as pl
from jax.experimental.pallas import tpu as pltpu
from jax.experimental.pallas import tpu_sc as plsc

_NW = 32
_LANES = 16


def _sc_compiler_params():
    cp = pltpu.CompilerParams()
    if "needs_layout_passes" in pltpu.CompilerParams.__dataclass_fields__:
        cp = dataclasses.replace(cp, needs_layout_passes=False)
    return cp


def _sc_mesh():
    return plsc.VectorSubcoreMesh(core_axis_name="c", subcore_axis_name="s")


def _sload(ref, i):
    return ref[pl.ds(i, _LANES)][0]


def _sc_seg_means(starts, ends, x, kmax):
    B = ends.shape[0]
    N, D = x.shape
    TB = B // _NW
    CB = 32
    WIN = CB * kmax + 8
    NCH = TB // CB
    NC = D // _LANES

    @functools.partial(
        pl.kernel,
        mesh=_sc_mesh(),
        compiler_params=_sc_compiler_params(),
        out_type=jax.ShapeDtypeStruct((B, D), jnp.float32),
        scratch_types=[
            pltpu.VMEM((WIN, D), jnp.float32),
            pltpu.VMEM((CB, D), jnp.float32),
            pltpu.VMEM((TB + 16,), jnp.int32),
            pltpu.VMEM((TB + 16,), jnp.int32),
            pltpu.SemaphoreType.DMA,
        ],
    )
    def k(starts_hbm, ends_hbm, x_hbm, m_hbm, xw, mbuf, starts_v, ends_v, sem):
        w = lax.axis_index("s") * 2 + lax.axis_index("c")
        pltpu.sync_copy(starts_hbm.at[pl.ds(w * TB, TB)], starts_v.at[pl.ds(0, TB)])
        pltpu.sync_copy(ends_hbm.at[pl.ds(w * TB, TB)], ends_v.at[pl.ds(0, TB)])

        @pl.loop(0, NCH)
        def _(ch):
            c0 = ch * CB
            s0 = jnp.minimum((_sload(starts_v, c0) // 8) * 8, N - WIN)
            pltpu.async_copy(x_hbm.at[pl.ds(s0, WIN)], xw, sem).wait()

            @pl.loop(0, CB)
            def _(bi):
                sti = _sload(starts_v, c0 + bi)
                st = sti - s0
                kk = _sload(ends_v, c0 + bi) - sti
                accs = [jnp.zeros((_LANES,), jnp.float32) for _ in range(NC)]
                for a in range(kmax):
                    valid = (a < kk).astype(jnp.float32)
                    for c in range(NC):
                        accs[c] = accs[c] + xw[st + a, pl.ds(c * _LANES, _LANES)] * valid
                inv = jnp.float32(0.0)
                for kv in range(1, kmax + 1):
                    inv = jnp.where(kk == kv, jnp.float32(1.0 / kv), inv)
                for c in range(NC):
                    mbuf[bi, pl.ds(c * _LANES, _LANES)] = accs[c] * inv

            pltpu.async_copy(mbuf, m_hbm.at[pl.ds(w * TB + c0, CB)], sem).wait()

    return k(starts, ends, x)


def _tc_matmul(m, w_neigh):
    B, D = m.shape
    BLK = 512

    def body(m_ref, w_ref, y_ref):
        y_ref[...] = jnp.dot(m_ref[...], w_ref[...],
                             preferred_element_type=jnp.float32)

    return pl.pallas_call(
        body,
        grid=(B // BLK,),
        in_specs=[pl.BlockSpec((BLK, D), lambda j: (j, 0)),
                  pl.BlockSpec((D, D), lambda j: (0, 0))],
        out_specs=pl.BlockSpec((BLK, D), lambda j: (j, 0)),
        out_shape=jax.ShapeDtypeStruct((B, D), jnp.float32),
    )(m, w_neigh)


def _sc_expand(y_ext, ends_ext, n_rows, b_segs, d_dim):
    D = d_dim
    N = n_rows
    B = b_segs
    TB = B // _NW
    YW = TB + 16
    RC = 256
    NC = D // _LANES

    @functools.partial(
        pl.kernel,
        mesh=_sc_mesh(),
        compiler_params=_sc_compiler_params(),
        out_type=jax.ShapeDtypeStruct((N * D,), jnp.float32),
        scratch_types=[
            pltpu.VMEM((YW, D), jnp.float32),
            pltpu.VMEM((RC * D,), jnp.float32),
            pltpu.VMEM((YW + 32,), jnp.int32),
            pltpu.SemaphoreType.DMA,
        ],
    )
    def k(y_hbm, ends_hbm, ya_hbm, yloc, obuf, ends_l, sem):
        w = lax.axis_index("s") * 2 + lax.axis_index("c")
        pltpu.sync_copy(y_hbm.at[pl.ds(w * TB, YW)], yloc)
        pltpu.sync_copy(ends_hbm.at[pl.ds(w * TB, YW + 16)],
                        ends_l.at[pl.ds(0, YW + 16)])
        r_lo = _sload(ends_l, 7)
        r_hi = jnp.where(w == _NW - 1, N, _sload(ends_l, TB + 7))
        nrows = r_hi - r_lo

        def copy_row(rl, r, t, et, buf):
            def adv(c):
                tt, _ = c
                return (tt + 1, _sload(ends_l, tt + 1))

            t, et = lax.while_loop(lambda c: r >= c[1], adv, (t, et))
            for c in range(NC):
                buf[pl.ds(rl * D + c * _LANES, _LANES)] = (
                    yloc[t, pl.ds(c * _LANES, _LANES)])
            return t, et

        def run(nrl, base_row, carry):
            def row(rl, c):
                t, et = c
                return copy_row(rl, base_row + rl, t, et, obuf)

            t, et = lax.fori_loop(0, nrl, row, carry)
            pltpu.async_copy(obuf.at[pl.ds(0, nrl * D)],
                             ya_hbm.at[pl.ds(base_row * D, nrl * D)],
                             sem).wait()
            return t, et

        t0 = jnp.int32(8)
        carry = (t0, _sload(ends_l, t0))
        nfull = nrows // RC
        carry = lax.fori_loop(
            0, nfull, lambda ci, c: run(RC, r_lo + ci * RC, c), carry)
        rem = nrows - nfull * RC
        base8 = r_lo + nfull * RC
        carry = lax.fori_loop(
            0, rem // 8, lambda j, c: run(8, base8 + j * 8, c), carry)
        base1 = base8 + (rem // 8) * 8
        lax.fori_loop(
            0, rem % 8, lambda j, c: run(1, base1 + j, c), carry)

    return k(y_ext, ends_ext)


def _tc_final(x, w_self, b, ya):
    N, D = x.shape
    BLK = 512

    def body(x_ref, w_ref, b_ref, ya_ref, o_ref):
        z = jnp.dot(x_ref[...], w_ref[...], preferred_element_type=jnp.float32)
        o_ref[...] = jnp.maximum(z + ya_ref[...] + b_ref[...], 0.0)

    return pl.pallas_call(
        body,
        grid=(N // BLK,),
        in_specs=[pl.BlockSpec((BLK, D), lambda j: (j, 0)),
                  pl.BlockSpec((D, D), lambda j: (0, 0)),
                  pl.BlockSpec((1, D), lambda j: (0, 0)),
                  pl.BlockSpec((BLK, D), lambda j: (j, 0))],
        out_specs=pl.BlockSpec((BLK, D), lambda j: (j, 0)),
        out_shape=jax.ShapeDtypeStruct((N, D), jnp.float32),
    )(x, w_self, b.reshape(1, D), ya)


def kernel(num_obj, object_branch_output, W_self, W_neigh, b):
    x = object_branch_output
    N, D = x.shape
    B = num_obj.shape[0]
    kmax = N // B
    counts = num_obj.astype(jnp.int32)
    ends = jnp.cumsum(counts)
    starts = ends - counts
    m = _sc_seg_means(starts, ends, x, kmax)
    y = _tc_matmul(m, W_neigh)
    zeros8 = jnp.zeros((8, D), jnp.float32)
    y_ext = jnp.concatenate([zeros8, y, zeros8], axis=0)
    ends_ext = jnp.concatenate(
        [jnp.zeros((8,), jnp.int32), ends, jnp.full((32,), N, jnp.int32)])
    ya = _sc_expand(y_ext, ends_ext, N, B, D).reshape(N, D)
    return _tc_final(x, W_self, b, ya)


# speedup vs baseline: 21.5903x; 6.6428x over previous
import dataclasses
import functools

import jax
import jax.numpy as jnp
from jax import lax
from jax.experimental import pallas as pl
from jax.experimental.pallas import tpu as pltpu
from jax.experimental.pallas import tpu_sc as plsc

_NW = 32
_LANES = 16


def _sc_compiler_params():
    cp = pltpu.CompilerParams()
    if "needs_layout_passes" in pltpu.CompilerParams.__dataclass_fields__:
        cp = dataclasses.replace(cp, needs_layout_passes=False)
    return cp


def _sc_mesh():
    return plsc.VectorSubcoreMesh(core_axis_name="c", subcore_axis_name="s")


def _sload(ref, i):
    return ref[pl.ds(i, _LANES)][0]


def _sc_seg_means(starts, ends, x, kmax):
    B = ends.shape[0]
    N, D = x.shape
    TB = B // _NW
    CB = 32
    WIN = CB * kmax + 8
    NCH = TB // CB
    NC = D // _LANES

    @functools.partial(
        pl.kernel,
        mesh=_sc_mesh(),
        compiler_params=_sc_compiler_params(),
        out_type=jax.ShapeDtypeStruct((B, D), jnp.float32),
        scratch_types=[
            pltpu.VMEM((WIN, D), jnp.float32),
            pltpu.VMEM((CB, D), jnp.float32),
            pltpu.VMEM((TB + 16,), jnp.int32),
            pltpu.VMEM((TB + 16,), jnp.int32),
            pltpu.SemaphoreType.DMA,
        ],
    )
    def k(starts_hbm, ends_hbm, x_hbm, m_hbm, xw, mbuf, starts_v, ends_v, sem):
        w = lax.axis_index("s") * 2 + lax.axis_index("c")
        pltpu.sync_copy(starts_hbm.at[pl.ds(w * TB, TB)], starts_v.at[pl.ds(0, TB)])
        pltpu.sync_copy(ends_hbm.at[pl.ds(w * TB, TB)], ends_v.at[pl.ds(0, TB)])

        @pl.loop(0, NCH)
        def _(ch):
            c0 = ch * CB
            s0 = jnp.minimum((_sload(starts_v, c0) // 8) * 8, N - WIN)
            pltpu.async_copy(x_hbm.at[pl.ds(s0, WIN)], xw, sem).wait()

            @pl.loop(0, CB)
            def _(bi):
                sti = _sload(starts_v, c0 + bi)
                st = sti - s0
                kk = _sload(ends_v, c0 + bi) - sti
                accs = [jnp.zeros((_LANES,), jnp.float32) for _ in range(NC)]
                for a in range(kmax):
                    valid = (a < kk).astype(jnp.float32)
                    for c in range(NC):
                        accs[c] = accs[c] + xw[st + a, pl.ds(c * _LANES, _LANES)] * valid
                inv = jnp.float32(0.0)
                for kv in range(1, kmax + 1):
                    inv = jnp.where(kk == kv, jnp.float32(1.0 / kv), inv)
                for c in range(NC):
                    mbuf[bi, pl.ds(c * _LANES, _LANES)] = accs[c] * inv

            pltpu.async_copy(mbuf, m_hbm.at[pl.ds(w * TB + c0, CB)], sem).wait()

    return k(starts, ends, x)


def _tc_matmul(m, w_neigh):
    B, D = m.shape
    BLK = 512

    def body(m_ref, w_ref, y_ref):
        y_ref[...] = jnp.dot(m_ref[...], w_ref[...],
                             preferred_element_type=jnp.float32)

    return pl.pallas_call(
        body,
        grid=(B // BLK,),
        in_specs=[pl.BlockSpec((BLK, D), lambda j: (j, 0)),
                  pl.BlockSpec((D, D), lambda j: (0, 0))],
        out_specs=pl.BlockSpec((BLK, D), lambda j: (j, 0)),
        out_shape=jax.ShapeDtypeStruct((B, D), jnp.float32),
    )(m, w_neigh)


def _sc_expand(y_ext, ends_ext, n_rows, b_segs, d_dim):
    D = d_dim
    N = n_rows
    B = b_segs
    TB = B // _NW
    YW = TB + 16
    RC = 256
    NC = D // _LANES

    @functools.partial(
        pl.kernel,
        mesh=_sc_mesh(),
        compiler_params=_sc_compiler_params(),
        out_type=jax.ShapeDtypeStruct((N, D), jnp.float32),
        scratch_types=[
            pltpu.VMEM((YW, D), jnp.float32),
            pltpu.VMEM((RC, D), jnp.float32),
            pltpu.VMEM((B + 56,), jnp.int32),
            pltpu.VMEM((32,), jnp.int32),
            pltpu.VMEM((8, D), jnp.float32),
            pltpu.SemaphoreType.DMA,
        ],
    )
    def k(y_hbm, ends_hbm, ya_hbm, yloc, obuf, ends_g, tvec_b, ybnd, sem):
        w = lax.axis_index("s") * 2 + lax.axis_index("c")
        pltpu.sync_copy(y_hbm.at[pl.ds(w * TB, YW)], yloc)
        pltpu.sync_copy(ends_hbm, ends_g.at[pl.ds(0, B + 40)])
        r_lo = _sload(ends_g, w * TB + 7)
        r_hi = _sload(ends_g, (w + 1) * TB + 7)
        a0 = (r_lo // 8) * 8
        a1 = jnp.where(w == _NW - 1, ((r_hi + 7) // 8) * 8, (r_hi // 8) * 8)
        nback = r_lo - a0

        lane = lax.iota(jnp.int32, _LANES)
        hi0 = jnp.full((_LANES,), B + 40, jnp.int32)
        lo0 = jnp.zeros((_LANES,), jnp.int32)

        def group(base_row):
            rvec = base_row + lane
            lo, hi = lo0, hi0
            for _ in range(15):
                mid = (lo + hi) >> 1
                ev = plsc.load_gather(ends_g, [mid])
                gt = ev > rvec
                lo = jnp.where(gt, lo, mid + 1)
                hi = jnp.where(gt, mid, hi)
            tvec_b[pl.ds(0, _LANES)] = lo - (w * TB)

        def chunk_rows(base_row):
            @pl.loop(0, RC // _LANES)
            def _(g):
                rl0 = g * _LANES
                group(base_row + rl0)
                for j in range(_LANES):
                    t = _sload(tvec_b, j)
                    t = jnp.minimum(jnp.maximum(t, 0), YW - 1)
                    for c in range(NC):
                        obuf[rl0 + j, pl.ds(c * _LANES, _LANES)] = (
                            yloc[t, pl.ds(c * _LANES, _LANES)])

        def fix_boundary():
            group(a0)
            for j in range(8):
                @pl.when(j < nback)
                def _():
                    jg = _sload(tvec_b, j) + w * TB
                    w8 = (jg // 8) * 8
                    pltpu.async_copy(y_hbm.at[pl.ds(w8, 8)], ybnd, sem).wait()
                    ri = jg - w8
                    for c in range(NC):
                        obuf[j, pl.ds(c * _LANES, _LANES)] = (
                            ybnd[ri, pl.ds(c * _LANES, _LANES)])

        nch = (N // _NW + RC) // RC

        def flush(base, limit):
            @pl.when(base + RC <= limit)
            def _():
                pltpu.async_copy(obuf, ya_hbm.at[pl.ds(base, RC)],
                                 sem).wait()

            @pl.when(base + RC > limit)
            def _():
                rem = limit - base
                off = jnp.int32(0)
                for sz in (128, 64, 32, 16, 8):
                    pred = (rem & sz) != 0
                    off_c = off

                    @pl.when(pred)
                    def _():
                        pltpu.async_copy(
                            obuf.at[pl.ds(off_c, sz)],
                            ya_hbm.at[pl.ds(base + off_c, sz)],
                            sem).wait()

                    off = off + jnp.where(pred, sz, 0)

        @pl.loop(0, nch)
        def _(ci):
            base = a0 + ci * RC

            @pl.when(base < a1)
            def _():
                chunk_rows(base)

                @pl.when(ci == 0)
                def _():
                    fix_boundary()

                flush(base, a1)

        total = _sload(ends_g, B + 7)
        z0 = ((total + 7) // 8) * 8
        nz8 = (N - z0) // 8
        zlo = z0 + (w * nz8 // _NW) * 8
        zhi = z0 + ((w + 1) * nz8 // _NW) * 8

        @pl.when(zlo < zhi)
        def _():
            zv = jnp.zeros((_LANES,), jnp.float32)

            @pl.loop(0, RC)
            def _(r):
                for c in range(NC):
                    obuf[r, pl.ds(c * _LANES, _LANES)] = zv

            @pl.loop(0, nch)
            def _(ci):
                base = zlo + ci * RC

                @pl.when(base < zhi)
                def _():
                    flush(base, zhi)

    return k(y_ext, ends_ext)


def _tc_final(x, w_self, b, ya):
    N, D = x.shape
    BLK = 512

    def body(x_ref, w_ref, b_ref, ya_ref, o_ref):
        z = jnp.dot(x_ref[...], w_ref[...], preferred_element_type=jnp.float32)
        o_ref[...] = jnp.maximum(z + ya_ref[...] + b_ref[...], 0.0)

    return pl.pallas_call(
        body,
        grid=(N // BLK,),
        in_specs=[pl.BlockSpec((BLK, D), lambda j: (j, 0)),
                  pl.BlockSpec((D, D), lambda j: (0, 0)),
                  pl.BlockSpec((1, D), lambda j: (0, 0)),
                  pl.BlockSpec((BLK, D), lambda j: (j, 0))],
        out_specs=pl.BlockSpec((BLK, D), lambda j: (j, 0)),
        out_shape=jax.ShapeDtypeStruct((N, D), jnp.float32),
    )(x, w_self, b.reshape(1, D), ya)


def kernel(num_obj, object_branch_output, W_self, W_neigh, b):
    x = object_branch_output
    N, D = x.shape
    B = num_obj.shape[0]
    kmax = N // B
    counts = num_obj.astype(jnp.int32)
    ends = jnp.cumsum(counts)
    starts = ends - counts
    m = _sc_seg_means(starts, ends, x, kmax)
    y = _tc_matmul(m, W_neigh)
    zeros8 = jnp.zeros((8, D), jnp.float32)
    y_ext = jnp.concatenate([zeros8, y, zeros8], axis=0)
    ends_ext = jnp.concatenate(
        [jnp.zeros((8,), jnp.int32), ends, jnp.full((32,), N, jnp.int32)])
    ya = _sc_expand(y_ext, ends_ext, N, B, D)
    return _tc_final(x, W_self, b, ya)
